# baseline (speedup 1.0000x reference)
import jax
import jax.numpy as jnp
from jax.experimental import pallas as pl
from jax.experimental.pallas import tpu as pltpu


def _make_kernel(d, s, a):
    cols_n = None

    def _kernel(w_hbm, c_hbm, o_hbm, ci_hbm, pat_ref, sem_w, sem_c, sem_i):
        n = ci_hbm.shape[1]
        copy_w = pltpu.make_async_copy(w_hbm, o_hbm.at[:, 0:d], sem_w)
        copy_c = pltpu.make_async_copy(c_hbm, o_hbm.at[:, d:2 * d], sem_c)
        copy_w.start()
        copy_c.start()

        r = jax.lax.broadcasted_iota(jnp.int32, (3, n), 0)
        j = jax.lax.broadcasted_iota(jnp.int32, (3, n), 1)
        bv = j // (s * a)
        q = j // a
        av = j - q * a
        sv = q - bv * s
        pat_ref[...] = jnp.where(r == 0, bv, jnp.where(r == 1, sv, av))
        copy_i = pltpu.make_async_copy(pat_ref, ci_hbm, sem_i)
        copy_i.start()

        copy_w.wait()
        copy_c.wait()
        copy_i.wait()

    return _kernel


def kernel(seq_mask, cnn_representation, word_representation,
           trigger_anchor_loc, trigger_anchor_labels, trigger_anchor_type,
           entity_candidates_repr, entity_candidates_mask,
           entity_candidates_len, entity_candidates_loc):
    B, S, D = word_representation.shape
    A = trigger_anchor_labels.shape[-1]
    N = B * S * A

    w2 = word_representation.reshape(B * S, D)
    c2 = cnn_representation.reshape(B * S, D)
    concat, cit = pl.pallas_call(
        _make_kernel(D, S, A),
        in_specs=[pl.BlockSpec(memory_space=pl.ANY),
                  pl.BlockSpec(memory_space=pl.ANY)],
        out_specs=[pl.BlockSpec(memory_space=pl.ANY),
                   pl.BlockSpec(memory_space=pl.ANY)],
        out_shape=[jax.ShapeDtypeStruct((B * S, 2 * D), jnp.float32),
                   jax.ShapeDtypeStruct((3, N), jnp.int32)],
        scratch_shapes=[pltpu.VMEM((3, N), jnp.int32),
                        pltpu.SemaphoreType.DMA,
                        pltpu.SemaphoreType.DMA,
                        pltpu.SemaphoreType.DMA],
    )(w2, c2)
    reg = concat.reshape(B, S, 2 * D)
    ci = cit.T

    zero_loss = jnp.zeros([1], jnp.float32)
    zero_label = jnp.zeros([B, S, A], jnp.int32)
    return (zero_loss, zero_label, zero_loss, zero_label, reg, ci)


# speedup vs baseline: 45.8994x; 45.8994x over previous
import jax
import jax.numpy as jnp
from jax.experimental import pallas as pl
from jax.experimental.pallas import tpu as pltpu


def _make_kernel(s, a, k):
    cols = k * s * a

    def _kernel(w_ref, c_ref, o_ref, ci_ref, pat_ref):
        d = w_ref.shape[1]
        o_ref[:, :d] = w_ref[...]
        o_ref[:, d:] = c_ref[...]

        i = pl.program_id(0)

        @pl.when(i == 0)
        def _():
            r = jax.lax.broadcasted_iota(jnp.int32, (3, cols), 0)
            j = jax.lax.broadcasted_iota(jnp.int32, (3, cols), 1)
            q = j // a
            av = j - q * a
            bv = q // s
            sv = q - bv * s
            pat_ref[...] = jnp.where(r == 0, bv, jnp.where(r == 1, sv, av))

        r = jax.lax.broadcasted_iota(jnp.int32, (3, cols), 0)
        ci_ref[...] = pat_ref[...] + jnp.where(r == 0, i * k, 0)

    return _kernel


def kernel(seq_mask, cnn_representation, word_representation,
           trigger_anchor_loc, trigger_anchor_labels, trigger_anchor_type,
           entity_candidates_repr, entity_candidates_mask,
           entity_candidates_len, entity_candidates_loc):
    B, S, D = word_representation.shape
    A = trigger_anchor_labels.shape[-1]
    N = B * S * A
    K = 2

    w2 = word_representation.reshape(B * S, D)
    c2 = cnn_representation.reshape(B * S, D)
    concat, cit = pl.pallas_call(
        _make_kernel(S, A, K),
        grid=(B // K,),
        in_specs=[pl.BlockSpec((K * S, D), lambda i: (i, 0)),
                  pl.BlockSpec((K * S, D), lambda i: (i, 0))],
        out_specs=[pl.BlockSpec((K * S, 2 * D), lambda i: (i, 0)),
                   pl.BlockSpec((3, K * S * A), lambda i: (0, i))],
        out_shape=[jax.ShapeDtypeStruct((B * S, 2 * D), jnp.float32),
                   jax.ShapeDtypeStruct((3, N), jnp.int32)],
        scratch_shapes=[pltpu.VMEM((3, K * S * A), jnp.int32)],
    )(w2, c2)
    reg = concat.reshape(B, S, 2 * D)
    ci = cit.T

    zero_loss = jnp.zeros([1], jnp.float32)
    zero_label = jnp.zeros([B, S, A], jnp.int32)
    return (zero_loss, zero_label, zero_loss, zero_label, reg, ci)
